# Initial kernel scaffold; baseline (speedup 1.0000x reference)
import functools

import jax
import jax.numpy as jnp
from jax import lax
from jax.experimental import pallas as pl
from jax.experimental.pallas import tpu as pltpu
from jax.experimental.pallas import tpu_sc as plsc

_VOCAB = 1000000
_D = 32
_OUT = 1000
_B = 16384
_H = 200

_NC = 2
_NS = 16
_NW = _NC * _NS
_RPW = _B // _NW
_C = 4
_IPC = _C * _H
_G = 80
_NG = _IPC // _G
_NCHUNK = _RPW // _C


def _pool_body(xf_hbm, tab_hbm, out_hbm, idx_v, rows_v, acc_v, gsem):
    wid = lax.axis_index("s") * _NC + lax.axis_index("c")
    row0 = wid * _RPW

    def chunk_body(g, carry):
        base_row = row0 + g * _C
        base_i = base_row * _H
        pltpu.sync_copy(xf_hbm.at[pl.ds(base_i, _IPC)], idx_v)
        copies = [
            pltpu.async_copy(
                tab_hbm.at[idx_v.at[pl.ds(k * _G, _G)]],
                rows_v.at[pl.ds(k * _G, _G)],
                gsem,
            )
            for k in range(_NG)
        ]
        for cp in copies:
            cp.wait()

        inv = jnp.float32(1.0 / _H)
        for c in range(_C):
            def red(j, accs):
                a0, a1, b0, b1 = accs
                r = c * _H + 2 * j
                a0 = a0 + rows_v[r, pl.ds(0, 16)]
                a1 = a1 + rows_v[r, pl.ds(16, 16)]
                b0 = b0 + rows_v[r + 1, pl.ds(0, 16)]
                b1 = b1 + rows_v[r + 1, pl.ds(16, 16)]
                return (a0, a1, b0, b1)

            z = jnp.zeros((16,), jnp.float32)
            a0, a1, b0, b1 = lax.fori_loop(0, _H // 2, red, (z, z, z, z))
            acc_v[c, pl.ds(0, 16)] = (a0 + b0) * inv
            acc_v[c, pl.ds(16, 16)] = (a1 + b1) * inv

        pltpu.sync_copy(acc_v, out_hbm.at[pl.ds(base_row, _C)])
        return carry

    lax.fori_loop(0, _NCHUNK, chunk_body, 0)


def _pool(x_flat, emb_table):
    mesh = plsc.VectorSubcoreMesh(core_axis_name="c", subcore_axis_name="s")
    return pl.kernel(
        _pool_body,
        out_type=jax.ShapeDtypeStruct((_B, _D), jnp.float32),
        mesh=mesh,
        scratch_types=[
            pltpu.VMEM((_IPC,), jnp.int32),
            pltpu.VMEM((_IPC, _D), jnp.float32),
            pltpu.VMEM((_C, _D), jnp.float32),
            pltpu.SemaphoreType.DMA,
        ],
    )(x_flat, emb_table)


def _mm_body(p_ref, w_ref, b_ref, o_ref):
    o_ref[...] = (
        jnp.dot(p_ref[...], w_ref[...], preferred_element_type=jnp.float32)
        + b_ref[...]
    )


def _matmul(pooled, w_t, bias):
    bm = 2048
    return pl.pallas_call(
        _mm_body,
        grid=(_B // bm,),
        in_specs=[
            pl.BlockSpec((bm, _D), lambda i: (i, 0)),
            pl.BlockSpec((_D, _OUT), lambda i: (0, 0)),
            pl.BlockSpec((1, _OUT), lambda i: (0, 0)),
        ],
        out_specs=pl.BlockSpec((bm, _OUT), lambda i: (i, 0)),
        out_shape=jax.ShapeDtypeStruct((_B, _OUT), jnp.float32),
    )(pooled, w_t, bias)


def kernel(x, emb_table, fc_w, fc_b):
    pooled = _pool(x.reshape(-1), emb_table)
    return _matmul(pooled, fc_w.T, fc_b.reshape(1, _OUT))


# speedup vs baseline: 10.9010x; 10.9010x over previous
import functools

import jax
import jax.numpy as jnp
from jax import lax
from jax.experimental import pallas as pl
from jax.experimental.pallas import tpu as pltpu
from jax.experimental.pallas import tpu_sc as plsc

_VOCAB = 1000000
_D = 32
_OUT = 1000
_B = 16384
_H = 200

_NC = 2
_NS = 16
_NW = _NC * _NS
_RPW = _B // _NW
_C = 4
_IPC = _C * _H
_G = 80
_NG = _IPC // _G
_NCHUNK = _RPW // _C


def _pool_body(xf_hbm, tab_hbm, out_hbm, idx_v, rows_v, acc_v, gsem):
    wid = lax.axis_index("s") * _NC + lax.axis_index("c")
    row0 = wid * _RPW

    def chunk_body(g, carry):
        base_row = row0 + g * _C
        base_i = base_row * _H
        pltpu.sync_copy(xf_hbm.at[pl.ds(base_i, _IPC)], idx_v)
        copies = [
            pltpu.async_copy(
                tab_hbm.at[idx_v.at[pl.ds(k * _G, _G)]],
                rows_v.at[pl.ds(k * _G, _G)],
                gsem,
            )
            for k in range(_NG)
        ]
        for cp in copies:
            cp.wait()

        inv = jnp.float32(1.0 / _H)
        for c in range(_C):
            def red(j, accs):
                a0, a1, b0, b1 = accs
                r = c * _H + 2 * j
                a0 = a0 + rows_v[r, pl.ds(0, 16)]
                a1 = a1 + rows_v[r, pl.ds(16, 16)]
                b0 = b0 + rows_v[r + 1, pl.ds(0, 16)]
                b1 = b1 + rows_v[r + 1, pl.ds(16, 16)]
                return (a0, a1, b0, b1)

            z = jnp.zeros((16,), jnp.float32)
            a0, a1, b0, b1 = lax.fori_loop(0, _H // 2, red, (z, z, z, z))
            acc_v[c, pl.ds(0, 16)] = (a0 + b0) * inv
            acc_v[c, pl.ds(16, 16)] = (a1 + b1) * inv

        pltpu.sync_copy(acc_v, out_hbm.at[pl.ds(base_row, _C)])
        return carry

    lax.fori_loop(0, _NCHUNK, chunk_body, 0)


def _pool(x_flat, emb_table):
    mesh = plsc.VectorSubcoreMesh(core_axis_name="c", subcore_axis_name="s")
    return pl.kernel(
        _pool_body,
        out_type=jax.ShapeDtypeStruct((_B, _D), jnp.float32),
        mesh=mesh,
        scratch_types=[
            pltpu.VMEM((_IPC,), jnp.int32),
            pltpu.VMEM((_IPC, _D), jnp.float32),
            pltpu.VMEM((_C, _D), jnp.float32),
            pltpu.SemaphoreType.DMA,
        ],
        compiler_params=pltpu.CompilerParams(use_tc_tiling_on_sc=False),
    )(x_flat, emb_table)


def _mm_body(p_ref, w_ref, b_ref, o_ref):
    o_ref[...] = (
        jnp.dot(p_ref[...], w_ref[...], preferred_element_type=jnp.float32)
        + b_ref[...]
    )


def _matmul(pooled, w_t, bias):
    bm = 2048
    return pl.pallas_call(
        _mm_body,
        grid=(_B // bm,),
        in_specs=[
            pl.BlockSpec((bm, _D), lambda i: (i, 0)),
            pl.BlockSpec((_D, _OUT), lambda i: (0, 0)),
            pl.BlockSpec((1, _OUT), lambda i: (0, 0)),
        ],
        out_specs=pl.BlockSpec((bm, _OUT), lambda i: (i, 0)),
        out_shape=jax.ShapeDtypeStruct((_B, _OUT), jnp.float32),
    )(pooled, w_t, bias)


def kernel(x, emb_table, fc_w, fc_b):
    pooled = _pool(x.reshape(-1), emb_table)
    return _matmul(pooled, fc_w.T, fc_b.reshape(1, _OUT))


# speedup vs baseline: 15.3352x; 1.4068x over previous
import jax
import jax.numpy as jnp
from jax import lax
from jax.experimental import pallas as pl
from jax.experimental.pallas import tpu as pltpu
from jax.experimental.pallas import tpu_sc as plsc

_VOCAB = 1000000
_D = 32
_OUT = 1000
_B = 16384
_H = 200

_NC = 2
_NS = 16
_NW = _NC * _NS
_RPW = _B // _NW
_C = 8
_G1 = 104
_G2 = _H - _G1
_NCHUNK = _RPW // _C
_NPAIR = _NCHUNK // 2


def _fire_gathers(tab, idx_b, rows_b, sem):
    for c in range(_C):
        pltpu.async_copy(
            tab.at[idx_b.at[c, pl.ds(0, _G1)]],
            rows_b.at[pl.ds(c * _H, _G1)], sem)
        pltpu.async_copy(
            tab.at[idx_b.at[c, pl.ds(_G1, _G2)]],
            rows_b.at[pl.ds(c * _H + _G1, _G2)], sem)


def _wait_gathers(tab, idx_b, rows_b, sem):
    for c in range(_C):
        pltpu.make_async_copy(
            tab.at[idx_b.at[c, pl.ds(0, _G1)]],
            rows_b.at[pl.ds(c * _H, _G1)], sem).wait()
        pltpu.make_async_copy(
            tab.at[idx_b.at[c, pl.ds(_G1, _G2)]],
            rows_b.at[pl.ds(c * _H + _G1, _G2)], sem).wait()


def _wait_idx(x_hbm, idx_b, sem):
    pltpu.make_async_copy(x_hbm.at[pl.ds(0, _C)], idx_b, sem).wait()


def _reduce_chunk(rows_b, stage_v, chunk):
    inv = jnp.float32(1.0 / _H)
    for c in range(_C):
        s0 = c * _H

        def body(j, accs, s0=s0):
            a0, a1, b0, b1 = accs
            r = s0 + 4 * j
            a0 = a0 + rows_b[r, pl.ds(0, 16)]
            a1 = a1 + rows_b[r, pl.ds(16, 16)]
            b0 = b0 + rows_b[r + 1, pl.ds(0, 16)]
            b1 = b1 + rows_b[r + 1, pl.ds(16, 16)]
            a0 = a0 + rows_b[r + 2, pl.ds(0, 16)]
            a1 = a1 + rows_b[r + 2, pl.ds(16, 16)]
            b0 = b0 + rows_b[r + 3, pl.ds(0, 16)]
            b1 = b1 + rows_b[r + 3, pl.ds(16, 16)]
            return (a0, a1, b0, b1)

        z = jnp.zeros((16,), jnp.float32)
        a0, a1, b0, b1 = lax.fori_loop(0, _H // 4, body, (z, z, z, z))
        slot = chunk * _C + c
        stage_v[slot, pl.ds(0, 16)] = (a0 + b0) * inv
        stage_v[slot, pl.ds(16, 16)] = (a1 + b1) * inv


def _pool_body(x_hbm, tab_hbm, out_hbm,
               idx0, idx1, rows0, rows1, stage_v, isem, gsem0, gsem1):
    wid = lax.axis_index("s") * _NC + lax.axis_index("c")
    row0 = wid * _RPW

    def idx_copy(t, dst):
        base = row0 + jnp.minimum(t, _NCHUNK - 1) * _C
        pltpu.async_copy(x_hbm.at[pl.ds(base, _C)], dst, isem)

    idx_copy(0, idx0)
    _wait_idx(x_hbm, idx0, isem)
    _fire_gathers(tab_hbm, idx0, rows0, gsem0)
    idx_copy(1, idx1)

    def pair_body(p, carry):
        a = 2 * p
        _wait_idx(x_hbm, idx1, isem)
        _fire_gathers(tab_hbm, idx1, rows1, gsem1)
        _wait_gathers(tab_hbm, idx0, rows0, gsem0)
        idx_copy(a + 2, idx0)
        _reduce_chunk(rows0, stage_v, a)
        _wait_idx(x_hbm, idx0, isem)
        _fire_gathers(tab_hbm, idx0, rows0, gsem0)
        _wait_gathers(tab_hbm, idx1, rows1, gsem1)
        idx_copy(a + 3, idx1)
        _reduce_chunk(rows1, stage_v, a + 1)
        return carry

    lax.fori_loop(0, _NPAIR, pair_body, 0)

    _wait_idx(x_hbm, idx1, isem)
    _wait_gathers(tab_hbm, idx0, rows0, gsem0)
    pltpu.sync_copy(stage_v, out_hbm.at[pl.ds(row0, _RPW)])


def _pool(x, emb_table):
    mesh = plsc.VectorSubcoreMesh(core_axis_name="c", subcore_axis_name="s")
    return pl.kernel(
        _pool_body,
        out_type=jax.ShapeDtypeStruct((_B, _D), jnp.float32),
        mesh=mesh,
        scratch_types=[
            pltpu.VMEM((_C, _H), jnp.int32),
            pltpu.VMEM((_C, _H), jnp.int32),
            pltpu.VMEM((_C * _H, _D), jnp.float32),
            pltpu.VMEM((_C * _H, _D), jnp.float32),
            pltpu.VMEM((_RPW, _D), jnp.float32),
            pltpu.SemaphoreType.DMA,
            pltpu.SemaphoreType.DMA,
            pltpu.SemaphoreType.DMA,
        ],
        compiler_params=pltpu.CompilerParams(use_tc_tiling_on_sc=False),
    )(x, emb_table)


def _mm_body(p_ref, w_ref, b_ref, o_ref):
    o_ref[...] = (
        jnp.dot(p_ref[...], w_ref[...], preferred_element_type=jnp.float32)
        + b_ref[...]
    )


def _matmul(pooled, w_t, bias):
    bm = 2048
    return pl.pallas_call(
        _mm_body,
        grid=(_B // bm,),
        in_specs=[
            pl.BlockSpec((bm, _D), lambda i: (i, 0)),
            pl.BlockSpec((_D, _OUT), lambda i: (0, 0)),
            pl.BlockSpec((1, _OUT), lambda i: (0, 0)),
        ],
        out_specs=pl.BlockSpec((bm, _OUT), lambda i: (i, 0)),
        out_shape=jax.ShapeDtypeStruct((_B, _OUT), jnp.float32),
    )(pooled, w_t, bias)


def kernel(x, emb_table, fc_w, fc_b):
    tab = emb_table.reshape(-1).reshape(_VOCAB, _D)
    pooled = _pool(x, tab)
    return _matmul(pooled, fc_w.T, fc_b.reshape(1, _OUT))
